# baseline (speedup 1.0000x reference)
import functools

import jax
import jax.numpy as jnp
from jax import lax
from jax.experimental import pallas as pl
from jax.experimental.pallas import tpu as pltpu
from jax.experimental.pallas import tpu_sc as plsc

N = 10000
E = 320000
D = 128
NUM_GRAPHS = 128
NUM_CLASSES = 10

NC = 2
NS = 16
NW = NC * NS
CHUNK = 128
CPT = 80
EPAD = NW * CPT * CHUNK
NPAD = 10112
ZR = NPAD // NS

NPOOL = 10240
PC = 64
RPT = NPOOL // NW
GP = 256
GZR = GP // NS

_mesh = plsc.VectorSubcoreMesh(
    core_axis_name="c", subcore_axis_name="s", num_cores=NC, num_subcores=NS)


@functools.partial(
    pl.kernel,
    out_type=jax.ShapeDtypeStruct((NC, NPAD, D), jnp.float32),
    mesh=_mesh,
    scratch_types=(
        pltpu.VMEM((CPT, CHUNK), jnp.int32),
        pltpu.VMEM((CPT, CHUNK), jnp.int32),
        pltpu.VMEM((CHUNK, D), jnp.float32),
        pltpu.VMEM_SHARED((NPAD, D), jnp.float32),
        pltpu.SemaphoreType.DMA,
    ),
)
def _sc_agg(h_hbm, src_hbm, dst_hbm, zeros_hbm,
            agg_out,
            src_v, dst_v, rows_v, acc_sh, sem):
    cid = lax.axis_index("c")
    sid = lax.axis_index("s")
    wid = sid * NC + cid
    pltpu.sync_copy(src_hbm.at[pl.ds(wid * CPT, CPT)], src_v)
    pltpu.sync_copy(dst_hbm.at[pl.ds(wid * CPT, CPT)], dst_v)
    pltpu.sync_copy(zeros_hbm.at[pl.ds(sid * ZR, ZR)],
                    acc_sh.at[pl.ds(sid * ZR, ZR)])
    plsc.subcore_barrier()

    def body(i, carry):
        pltpu.async_copy(h_hbm.at[src_v.at[i]], rows_v, sem).wait()
        pltpu.sync_copy(rows_v, acc_sh.at[dst_v.at[i]], add=True)
        return carry

    lax.fori_loop(0, CPT, body, 0)
    plsc.subcore_barrier()
    pltpu.sync_copy(acc_sh.at[pl.ds(sid * ZR, ZR)],
                    agg_out.at[cid, pl.ds(sid * ZR, ZR)])


@functools.partial(
    pl.kernel,
    out_type=jax.ShapeDtypeStruct((NC, NPAD, D), jnp.float32),
    mesh=_mesh,
    scratch_types=(
        pltpu.VMEM((CPT, CHUNK), jnp.int32),
        pltpu.VMEM((CHUNK, D), jnp.float32),
        pltpu.VMEM_SHARED((NPAD, D), jnp.float32),
    ),
)
def _sc_deg(dst_hbm, zeros_hbm, ones_hbm,
            deg_out,
            dst_v, ones_v, acc_sh):
    cid = lax.axis_index("c")
    sid = lax.axis_index("s")
    wid = sid * NC + cid
    pltpu.sync_copy(dst_hbm.at[pl.ds(wid * CPT, CPT)], dst_v)
    pltpu.sync_copy(zeros_hbm.at[pl.ds(sid * ZR, ZR)],
                    acc_sh.at[pl.ds(sid * ZR, ZR)])
    pltpu.sync_copy(ones_hbm, ones_v)
    plsc.subcore_barrier()

    def body(i, carry):
        pltpu.sync_copy(ones_v, acc_sh.at[dst_v.at[i]], add=True)
        return carry

    lax.fori_loop(0, CPT, body, 0)
    plsc.subcore_barrier()
    pltpu.sync_copy(acc_sh.at[pl.ds(sid * ZR, ZR)],
                    deg_out.at[cid, pl.ds(sid * ZR, ZR)])


@functools.partial(
    pl.kernel,
    out_type=jax.ShapeDtypeStruct((NC, GP, D), jnp.float32),
    mesh=_mesh,
    scratch_types=(
        pltpu.VMEM((RPT // PC, PC), jnp.int32),
        pltpu.VMEM((PC, D), jnp.float32),
        pltpu.VMEM_SHARED((GP, D), jnp.float32),
    ),
)
def _sc_pool(h_hbm, b_hbm, zeros_hbm,
             g_out,
             bidx_v, rows_v, acc_sh):
    cid = lax.axis_index("c")
    sid = lax.axis_index("s")
    wid = sid * NC + cid
    pltpu.sync_copy(b_hbm.at[wid], bidx_v)
    pltpu.sync_copy(zeros_hbm.at[pl.ds(sid * GZR, GZR)],
                    acc_sh.at[pl.ds(sid * GZR, GZR)])
    plsc.subcore_barrier()

    def body(i, carry):
        base = wid * RPT + i * PC
        pltpu.sync_copy(h_hbm.at[pl.ds(base, PC)], rows_v)
        pltpu.sync_copy(rows_v, acc_sh.at[bidx_v.at[i]], add=True)
        return carry

    lax.fori_loop(0, RPT // PC, body, 0)
    plsc.subcore_barrier()
    pltpu.sync_copy(acc_sh.at[pl.ds(sid * GZR, GZR)],
                    g_out.at[cid, pl.ds(sid * GZR, GZR)])


def _tc_layer_body(p_ref, d_ref, h_ref, wl_ref, bl_ref, wr_ref, o_ref):
    deg = jnp.maximum(d_ref[0, :, 0:1] + d_ref[1, :, 0:1], 1.0)
    agg = (p_ref[0] + p_ref[1]) / deg
    dn = (((1,), (1,)), ((), ()))
    y = (lax.dot_general(agg, wl_ref[...], dn, preferred_element_type=jnp.float32)
         + lax.dot_general(h_ref[...], wr_ref[...], dn, preferred_element_type=jnp.float32)
         + bl_ref[...])
    o_ref[...] = jnp.maximum(y, 0.0)


_BN = 1000


def _tc_layer(p, dpart, h, Wl, bl, Wr):
    return pl.pallas_call(
        _tc_layer_body,
        grid=(N // _BN,),
        in_specs=[
            pl.BlockSpec((NC, _BN, D), lambda i: (0, i, 0)),
            pl.BlockSpec((NC, _BN, D), lambda i: (0, i, 0)),
            pl.BlockSpec((_BN, D), lambda i: (i, 0)),
            pl.BlockSpec((D, D), lambda i: (0, 0)),
            pl.BlockSpec((1, D), lambda i: (0, 0)),
            pl.BlockSpec((D, D), lambda i: (0, 0)),
        ],
        out_specs=pl.BlockSpec((_BN, D), lambda i: (i, 0)),
        out_shape=jax.ShapeDtypeStruct((N, D), jnp.float32),
    )(p, dpart, h, Wl, bl.reshape(1, D), Wr)


def _tc_head_body(g_ref, w1_ref, b1_ref, w2_ref, b2_ref, o_ref):
    g = jnp.maximum(g_ref[0, :NUM_GRAPHS, :] + g_ref[1, :NUM_GRAPHS, :], 0.0)
    dn = (((1,), (1,)), ((), ()))
    y = jnp.maximum(
        lax.dot_general(g, w1_ref[...], dn, preferred_element_type=jnp.float32)
        + b1_ref[...], 0.0)
    z = (lax.dot_general(y, w2_ref[...], dn, preferred_element_type=jnp.float32)
         + b2_ref[...])
    m = jnp.max(z, axis=1, keepdims=True)
    zs = z - m
    lse = jnp.log(jnp.sum(jnp.exp(zs), axis=1, keepdims=True))
    o_ref[...] = zs - lse


def _tc_head(gp, W1, b1, W2, b2):
    return pl.pallas_call(
        _tc_head_body,
        in_specs=[
            pl.BlockSpec((NC, GP, D), lambda: (0, 0, 0)),
            pl.BlockSpec((D, D), lambda: (0, 0)),
            pl.BlockSpec((1, D), lambda: (0, 0)),
            pl.BlockSpec((NUM_CLASSES, D), lambda: (0, 0)),
            pl.BlockSpec((1, NUM_CLASSES), lambda: (0, 0)),
        ],
        out_specs=pl.BlockSpec((NUM_GRAPHS, NUM_CLASSES), lambda: (0, 0)),
        out_shape=jax.ShapeDtypeStruct((NUM_GRAPHS, NUM_CLASSES), jnp.float32),
    )(gp, W1, b1.reshape(1, D), W2, b2.reshape(1, NUM_CLASSES))


def kernel(x, edge_attr, edge_index, batch, Wl0, bl0, Wr0, Wl1, bl1, Wr1,
           Wl2, bl2, Wr2, W1, b1, W2, b2):
    del edge_attr
    src_p = jnp.concatenate(
        [edge_index[0], jnp.zeros((EPAD - E,), jnp.int32)]).reshape(
            EPAD // CHUNK, CHUNK)
    dst_p = jnp.concatenate(
        [edge_index[1], jnp.full((EPAD - E,), N, jnp.int32)]).reshape(
            EPAD // CHUNK, CHUNK)
    zeros_big = jnp.zeros((NPAD, D), jnp.float32)
    ones_c = jnp.ones((CHUNK, D), jnp.float32)

    dpart = _sc_deg(dst_p, zeros_big, ones_c)
    p0 = _sc_agg(x, src_p, dst_p, zeros_big)
    h1 = _tc_layer(p0, dpart, x, Wl0, bl0, Wr0)
    p1 = _sc_agg(h1, src_p, dst_p, zeros_big)
    h2 = _tc_layer(p1, dpart, h1, Wl1, bl1, Wr1)
    p2 = _sc_agg(h2, src_p, dst_p, zeros_big)
    h3 = _tc_layer(p2, dpart, h2, Wl2, bl2, Wr2)

    h3p = jnp.concatenate([h3, jnp.zeros((NPOOL - N, D), jnp.float32)], axis=0)
    b_p = jnp.concatenate(
        [batch, jnp.full((NPOOL - N,), NUM_GRAPHS, jnp.int32)]).reshape(
            NW, RPT // PC, PC)
    gp = _sc_pool(h3p, b_p, zeros_big)
    return _tc_head(gp, W1, b1, W2, b2)


# speedup vs baseline: 1.1371x; 1.1371x over previous
import functools

import jax
import jax.numpy as jnp
from jax import lax
from jax.experimental import pallas as pl
from jax.experimental.pallas import tpu as pltpu
from jax.experimental.pallas import tpu_sc as plsc

N = 10000
E = 320000
D = 128
NUM_GRAPHS = 128
NUM_CLASSES = 10

NC = 2
NS = 16
NW = NC * NS
CHUNK = 128
CPT = 80
EPAD = NW * CPT * CHUNK
NPAD = 10112
ZR = NPAD // NS

NPOOL = 10240
PC = 64
RPT = NPOOL // NW
GP = 256
GZR = GP // NS

_mesh = plsc.VectorSubcoreMesh(
    core_axis_name="c", subcore_axis_name="s", num_cores=NC, num_subcores=NS)


@functools.partial(
    pl.kernel,
    out_type=jax.ShapeDtypeStruct((NC, NPAD, D), jnp.float32),
    mesh=_mesh,
    scratch_types=(
        pltpu.VMEM((CHUNK,), jnp.int32),
        pltpu.VMEM((CHUNK,), jnp.int32),
        pltpu.VMEM((CHUNK,), jnp.int32),
        pltpu.VMEM((CHUNK,), jnp.int32),
        pltpu.VMEM((CHUNK, D), jnp.float32),
        pltpu.VMEM((CHUNK, D), jnp.float32),
        pltpu.VMEM_SHARED((NPAD, D), jnp.float32),
        pltpu.SemaphoreType.DMA,
        pltpu.SemaphoreType.DMA,
    ),
)
def _sc_agg(h_hbm, src_hbm, dst_hbm, zeros_hbm,
            agg_out,
            src_v0, dst_v0, src_v1, dst_v1, rows_v0, rows_v1,
            acc_sh, sem0, sem1):
    cid = lax.axis_index("c")
    sid = lax.axis_index("s")
    wid = sid * NC + cid
    pltpu.sync_copy(zeros_hbm.at[pl.ds(sid * ZR, ZR)],
                    acc_sh.at[pl.ds(sid * ZR, ZR)])
    plsc.subcore_barrier()

    def fire(c, sb, db, rb, sm):
        base = (wid * CPT + c) * CHUNK
        pltpu.sync_copy(src_hbm.at[pl.ds(base, CHUNK)], sb)
        pltpu.sync_copy(dst_hbm.at[pl.ds(base, CHUNK)], db)
        pltpu.async_copy(h_hbm.at[sb], rb, sm)

    def drain_scatter(db, rb, sm):
        pltpu.make_async_copy(zeros_hbm.at[pl.ds(0, CHUNK)], rb, sm).wait()
        pltpu.sync_copy(rb, acc_sh.at[db], add=True)

    fire(0, src_v0, dst_v0, rows_v0, sem0)
    fire(1, src_v1, dst_v1, rows_v1, sem1)

    def body(g, carry):
        drain_scatter(dst_v0, rows_v0, sem0)
        fire(2 * g + 2, src_v0, dst_v0, rows_v0, sem0)
        drain_scatter(dst_v1, rows_v1, sem1)
        fire(2 * g + 3, src_v1, dst_v1, rows_v1, sem1)
        return carry

    lax.fori_loop(0, (CPT - 2) // 2, body, 0)
    drain_scatter(dst_v0, rows_v0, sem0)
    drain_scatter(dst_v1, rows_v1, sem1)
    plsc.subcore_barrier()
    pltpu.sync_copy(acc_sh.at[pl.ds(sid * ZR, ZR)],
                    agg_out.at[cid, pl.ds(sid * ZR, ZR)])


@functools.partial(
    pl.kernel,
    out_type=jax.ShapeDtypeStruct((NC, NPAD, D), jnp.float32),
    mesh=_mesh,
    scratch_types=(
        pltpu.VMEM((CHUNK,), jnp.int32),
        pltpu.VMEM((CHUNK, D), jnp.float32),
        pltpu.VMEM_SHARED((NPAD, D), jnp.float32),
    ),
)
def _sc_deg(dst_hbm, zeros_hbm, ones_hbm,
            deg_out,
            dst_v, ones_v, acc_sh):
    cid = lax.axis_index("c")
    sid = lax.axis_index("s")
    wid = sid * NC + cid
    pltpu.sync_copy(zeros_hbm.at[pl.ds(sid * ZR, ZR)],
                    acc_sh.at[pl.ds(sid * ZR, ZR)])
    pltpu.sync_copy(ones_hbm, ones_v)
    plsc.subcore_barrier()

    def body(i, carry):
        base = (wid * CPT + i) * CHUNK
        pltpu.sync_copy(dst_hbm.at[pl.ds(base, CHUNK)], dst_v)
        pltpu.sync_copy(ones_v, acc_sh.at[dst_v], add=True)
        return carry

    lax.fori_loop(0, CPT, body, 0)
    plsc.subcore_barrier()
    pltpu.sync_copy(acc_sh.at[pl.ds(sid * ZR, ZR)],
                    deg_out.at[cid, pl.ds(sid * ZR, ZR)])


@functools.partial(
    pl.kernel,
    out_type=jax.ShapeDtypeStruct((NC, GP, D), jnp.float32),
    mesh=_mesh,
    scratch_types=(
        pltpu.VMEM((PC,), jnp.int32),
        pltpu.VMEM((PC, D), jnp.float32),
        pltpu.VMEM_SHARED((GP, D), jnp.float32),
    ),
)
def _sc_pool(h_hbm, b_hbm, zeros_hbm,
             g_out,
             bidx_v, rows_v, acc_sh):
    cid = lax.axis_index("c")
    sid = lax.axis_index("s")
    wid = sid * NC + cid
    pltpu.sync_copy(zeros_hbm.at[pl.ds(sid * GZR, GZR)],
                    acc_sh.at[pl.ds(sid * GZR, GZR)])
    plsc.subcore_barrier()

    def body(i, carry):
        base = wid * RPT + i * PC
        pltpu.sync_copy(b_hbm.at[pl.ds(base, PC)], bidx_v)
        pltpu.sync_copy(h_hbm.at[pl.ds(base, PC)], rows_v)
        pltpu.sync_copy(rows_v, acc_sh.at[bidx_v], add=True)
        return carry

    lax.fori_loop(0, RPT // PC, body, 0)
    plsc.subcore_barrier()
    pltpu.sync_copy(acc_sh.at[pl.ds(sid * GZR, GZR)],
                    g_out.at[cid, pl.ds(sid * GZR, GZR)])


def _tc_layer_body(p_ref, d_ref, h_ref, wl_ref, bl_ref, wr_ref, o_ref):
    deg = jnp.maximum(d_ref[0, :, 0:1] + d_ref[1, :, 0:1], 1.0)
    agg = (p_ref[0] + p_ref[1]) / deg
    dn = (((1,), (1,)), ((), ()))
    y = (lax.dot_general(agg, wl_ref[...], dn, preferred_element_type=jnp.float32)
         + lax.dot_general(h_ref[...], wr_ref[...], dn, preferred_element_type=jnp.float32)
         + bl_ref[...])
    o_ref[...] = jnp.maximum(y, 0.0)


_BN = 1000


def _tc_layer(p, dpart, h, Wl, bl, Wr):
    return pl.pallas_call(
        _tc_layer_body,
        grid=(N // _BN,),
        in_specs=[
            pl.BlockSpec((NC, _BN, D), lambda i: (0, i, 0)),
            pl.BlockSpec((NC, _BN, D), lambda i: (0, i, 0)),
            pl.BlockSpec((_BN, D), lambda i: (i, 0)),
            pl.BlockSpec((D, D), lambda i: (0, 0)),
            pl.BlockSpec((1, D), lambda i: (0, 0)),
            pl.BlockSpec((D, D), lambda i: (0, 0)),
        ],
        out_specs=pl.BlockSpec((_BN, D), lambda i: (i, 0)),
        out_shape=jax.ShapeDtypeStruct((N, D), jnp.float32),
    )(p, dpart, h, Wl, bl.reshape(1, D), Wr)


def _tc_head_body(g_ref, w1_ref, b1_ref, w2_ref, b2_ref, o_ref):
    g = jnp.maximum(g_ref[0, :NUM_GRAPHS, :] + g_ref[1, :NUM_GRAPHS, :], 0.0)
    dn = (((1,), (1,)), ((), ()))
    y = jnp.maximum(
        lax.dot_general(g, w1_ref[...], dn, preferred_element_type=jnp.float32)
        + b1_ref[...], 0.0)
    z = (lax.dot_general(y, w2_ref[...], dn, preferred_element_type=jnp.float32)
         + b2_ref[...])
    m = jnp.max(z, axis=1, keepdims=True)
    zs = z - m
    lse = jnp.log(jnp.sum(jnp.exp(zs), axis=1, keepdims=True))
    o_ref[...] = zs - lse


def _tc_head(gp, W1, b1, W2, b2):
    return pl.pallas_call(
        _tc_head_body,
        in_specs=[
            pl.BlockSpec((NC, GP, D), lambda: (0, 0, 0)),
            pl.BlockSpec((D, D), lambda: (0, 0)),
            pl.BlockSpec((1, D), lambda: (0, 0)),
            pl.BlockSpec((NUM_CLASSES, D), lambda: (0, 0)),
            pl.BlockSpec((1, NUM_CLASSES), lambda: (0, 0)),
        ],
        out_specs=pl.BlockSpec((NUM_GRAPHS, NUM_CLASSES), lambda: (0, 0)),
        out_shape=jax.ShapeDtypeStruct((NUM_GRAPHS, NUM_CLASSES), jnp.float32),
    )(gp, W1, b1.reshape(1, D), W2, b2.reshape(1, NUM_CLASSES))


def kernel(x, edge_attr, edge_index, batch, Wl0, bl0, Wr0, Wl1, bl1, Wr1,
           Wl2, bl2, Wr2, W1, b1, W2, b2):
    del edge_attr
    src_p = jnp.concatenate(
        [edge_index[0], jnp.zeros((EPAD - E,), jnp.int32)])
    dst_p = jnp.concatenate(
        [edge_index[1], jnp.full((EPAD - E,), N, jnp.int32)])
    zeros_big = jnp.zeros((NPAD, D), jnp.float32)
    ones_c = jnp.ones((CHUNK, D), jnp.float32)

    dpart = _sc_deg(dst_p, zeros_big, ones_c)
    p0 = _sc_agg(x, src_p, dst_p, zeros_big)
    h1 = _tc_layer(p0, dpart, x, Wl0, bl0, Wr0)
    p1 = _sc_agg(h1, src_p, dst_p, zeros_big)
    h2 = _tc_layer(p1, dpart, h1, Wl1, bl1, Wr1)
    p2 = _sc_agg(h2, src_p, dst_p, zeros_big)
    h3 = _tc_layer(p2, dpart, h2, Wl2, bl2, Wr2)

    h3p = jnp.concatenate([h3, jnp.zeros((NPOOL - N, D), jnp.float32)], axis=0)
    b_p = jnp.concatenate(
        [batch, jnp.full((NPOOL - N,), NUM_GRAPHS, jnp.int32)])
    gp = _sc_pool(h3p, b_p, zeros_big)
    return _tc_head(gp, W1, b1, W2, b2)


# speedup vs baseline: 1.3301x; 1.1697x over previous
import functools

import jax
import jax.numpy as jnp
from jax import lax
from jax.experimental import pallas as pl
from jax.experimental.pallas import tpu as pltpu
from jax.experimental.pallas import tpu_sc as plsc

N = 10000
E = 320000
D = 128
NUM_GRAPHS = 128
NUM_CLASSES = 10

NC = 2
NS = 16
NW = NC * NS
CHUNK = 128
CPT = 79
EPAD = NW * CPT * CHUNK
NPAD = 10112
ZR = NPAD // NS

NPOOL = 10240
PC = 64
RPT = NPOOL // NW
GP = 256
GZR = GP // NS

_mesh = plsc.VectorSubcoreMesh(
    core_axis_name="c", subcore_axis_name="s", num_cores=NC, num_subcores=NS)


@functools.partial(
    pl.kernel,
    out_type=jax.ShapeDtypeStruct((NC, NPAD, D), jnp.float32),
    mesh=_mesh,
    scratch_types=(
        pltpu.VMEM((CHUNK,), jnp.int32),
        pltpu.VMEM((CHUNK,), jnp.int32),
        pltpu.VMEM((CHUNK, D), jnp.float32),
        pltpu.VMEM_SHARED((NPAD, D), jnp.float32),
        pltpu.SemaphoreType.DMA,
    ),
)
def _sc_agg(h_hbm, src_hbm, dst_hbm, zeros_hbm,
            agg_out,
            src_v, dst_v, rows_v, acc_sh, sem):
    cid = lax.axis_index("c")
    sid = lax.axis_index("s")
    wid = sid * NC + cid
    pltpu.sync_copy(zeros_hbm.at[pl.ds(sid * ZR, ZR)],
                    acc_sh.at[pl.ds(sid * ZR, ZR)])
    plsc.subcore_barrier()

    def body(i, carry):
        base = (wid * CPT + i) * CHUNK
        pltpu.sync_copy(src_hbm.at[pl.ds(base, CHUNK)], src_v)
        pltpu.sync_copy(dst_hbm.at[pl.ds(base, CHUNK)], dst_v)
        pltpu.async_copy(h_hbm.at[src_v], rows_v, sem).wait()
        pltpu.sync_copy(rows_v, acc_sh.at[dst_v], add=True)
        return carry

    lax.fori_loop(0, CPT, body, 0)
    plsc.subcore_barrier()
    pltpu.sync_copy(acc_sh.at[pl.ds(sid * ZR, ZR)],
                    agg_out.at[cid, pl.ds(sid * ZR, ZR)])


@functools.partial(
    pl.kernel,
    out_type=jax.ShapeDtypeStruct((NC, NPAD, D), jnp.float32),
    mesh=_mesh,
    scratch_types=(
        pltpu.VMEM((CHUNK,), jnp.int32),
        pltpu.VMEM((CHUNK, D), jnp.float32),
        pltpu.VMEM_SHARED((NPAD, D), jnp.float32),
    ),
)
def _sc_deg(dst_hbm, zeros_hbm, ones_hbm,
            deg_out,
            dst_v, ones_v, acc_sh):
    cid = lax.axis_index("c")
    sid = lax.axis_index("s")
    wid = sid * NC + cid
    pltpu.sync_copy(zeros_hbm.at[pl.ds(sid * ZR, ZR)],
                    acc_sh.at[pl.ds(sid * ZR, ZR)])
    pltpu.sync_copy(ones_hbm, ones_v)
    plsc.subcore_barrier()

    def body(i, carry):
        base = (wid * CPT + i) * CHUNK
        pltpu.sync_copy(dst_hbm.at[pl.ds(base, CHUNK)], dst_v)
        pltpu.sync_copy(ones_v, acc_sh.at[dst_v], add=True)
        return carry

    lax.fori_loop(0, CPT, body, 0)
    plsc.subcore_barrier()
    pltpu.sync_copy(acc_sh.at[pl.ds(sid * ZR, ZR)],
                    deg_out.at[cid, pl.ds(sid * ZR, ZR)])


@functools.partial(
    pl.kernel,
    out_type=jax.ShapeDtypeStruct((NC, GP, D), jnp.float32),
    mesh=_mesh,
    scratch_types=(
        pltpu.VMEM((PC,), jnp.int32),
        pltpu.VMEM((PC, D), jnp.float32),
        pltpu.VMEM_SHARED((GP, D), jnp.float32),
    ),
)
def _sc_pool(h_hbm, b_hbm, zeros_hbm,
             g_out,
             bidx_v, rows_v, acc_sh):
    cid = lax.axis_index("c")
    sid = lax.axis_index("s")
    wid = sid * NC + cid
    pltpu.sync_copy(zeros_hbm.at[pl.ds(sid * GZR, GZR)],
                    acc_sh.at[pl.ds(sid * GZR, GZR)])
    plsc.subcore_barrier()

    def body(i, carry):
        base = wid * RPT + i * PC
        pltpu.sync_copy(b_hbm.at[pl.ds(base, PC)], bidx_v)
        pltpu.sync_copy(h_hbm.at[pl.ds(base, PC)], rows_v)
        pltpu.sync_copy(rows_v, acc_sh.at[bidx_v], add=True)
        return carry

    lax.fori_loop(0, RPT // PC, body, 0)
    plsc.subcore_barrier()
    pltpu.sync_copy(acc_sh.at[pl.ds(sid * GZR, GZR)],
                    g_out.at[cid, pl.ds(sid * GZR, GZR)])


def _tc_layer_body(p_ref, d_ref, h_ref, wl_ref, bl_ref, wr_ref, o_ref):
    deg = jnp.maximum(d_ref[0, :, 0:1] + d_ref[1, :, 0:1], 1.0)
    agg = (p_ref[0] + p_ref[1]) / deg
    dn = (((1,), (1,)), ((), ()))
    y = (lax.dot_general(agg, wl_ref[...], dn, preferred_element_type=jnp.float32)
         + lax.dot_general(h_ref[...], wr_ref[...], dn, preferred_element_type=jnp.float32)
         + bl_ref[...])
    o_ref[...] = jnp.maximum(y, 0.0)


_BN = 1000


def _tc_layer(p, dpart, h, Wl, bl, Wr):
    return pl.pallas_call(
        _tc_layer_body,
        grid=(N // _BN,),
        in_specs=[
            pl.BlockSpec((NC, _BN, D), lambda i: (0, i, 0)),
            pl.BlockSpec((NC, _BN, D), lambda i: (0, i, 0)),
            pl.BlockSpec((_BN, D), lambda i: (i, 0)),
            pl.BlockSpec((D, D), lambda i: (0, 0)),
            pl.BlockSpec((1, D), lambda i: (0, 0)),
            pl.BlockSpec((D, D), lambda i: (0, 0)),
        ],
        out_specs=pl.BlockSpec((_BN, D), lambda i: (i, 0)),
        out_shape=jax.ShapeDtypeStruct((N, D), jnp.float32),
    )(p, dpart, h, Wl, bl.reshape(1, D), Wr)


def _tc_head_body(g_ref, w1_ref, b1_ref, w2_ref, b2_ref, o_ref):
    g = jnp.maximum(g_ref[0, :NUM_GRAPHS, :] + g_ref[1, :NUM_GRAPHS, :], 0.0)
    dn = (((1,), (1,)), ((), ()))
    y = jnp.maximum(
        lax.dot_general(g, w1_ref[...], dn, preferred_element_type=jnp.float32)
        + b1_ref[...], 0.0)
    z = (lax.dot_general(y, w2_ref[...], dn, preferred_element_type=jnp.float32)
         + b2_ref[...])
    m = jnp.max(z, axis=1, keepdims=True)
    zs = z - m
    lse = jnp.log(jnp.sum(jnp.exp(zs), axis=1, keepdims=True))
    o_ref[...] = zs - lse


def _tc_head(gp, W1, b1, W2, b2):
    return pl.pallas_call(
        _tc_head_body,
        in_specs=[
            pl.BlockSpec((NC, GP, D), lambda: (0, 0, 0)),
            pl.BlockSpec((D, D), lambda: (0, 0)),
            pl.BlockSpec((1, D), lambda: (0, 0)),
            pl.BlockSpec((NUM_CLASSES, D), lambda: (0, 0)),
            pl.BlockSpec((1, NUM_CLASSES), lambda: (0, 0)),
        ],
        out_specs=pl.BlockSpec((NUM_GRAPHS, NUM_CLASSES), lambda: (0, 0)),
        out_shape=jax.ShapeDtypeStruct((NUM_GRAPHS, NUM_CLASSES), jnp.float32),
    )(gp, W1, b1.reshape(1, D), W2, b2.reshape(1, NUM_CLASSES))


def kernel(x, edge_attr, edge_index, batch, Wl0, bl0, Wr0, Wl1, bl1, Wr1,
           Wl2, bl2, Wr2, W1, b1, W2, b2):
    del edge_attr
    src_p = jnp.concatenate(
        [edge_index[0], jnp.zeros((EPAD - E,), jnp.int32)])
    dst_p = jnp.concatenate(
        [edge_index[1], jnp.full((EPAD - E,), N, jnp.int32)])
    zeros_big = jnp.zeros((NPAD, D), jnp.float32)
    ones_c = jnp.ones((CHUNK, D), jnp.float32)

    dpart = _sc_deg(dst_p, zeros_big, ones_c)
    p0 = _sc_agg(x, src_p, dst_p, zeros_big)
    h1 = _tc_layer(p0, dpart, x, Wl0, bl0, Wr0)
    p1 = _sc_agg(h1, src_p, dst_p, zeros_big)
    h2 = _tc_layer(p1, dpart, h1, Wl1, bl1, Wr1)
    p2 = _sc_agg(h2, src_p, dst_p, zeros_big)
    h3 = _tc_layer(p2, dpart, h2, Wl2, bl2, Wr2)

    h3p = jnp.concatenate([h3, jnp.zeros((NPOOL - N, D), jnp.float32)], axis=0)
    b_p = jnp.concatenate(
        [batch, jnp.full((NPOOL - N,), NUM_GRAPHS, jnp.int32)])
    gp = _sc_pool(h3p, b_p, zeros_big)
    return _tc_head(gp, W1, b1, W2, b2)
